# Initial kernel scaffold; baseline (speedup 1.0000x reference)
import jax
import jax.numpy as jnp
from jax.experimental import pallas as pl


def kernel(words, feats, word_table, feat_table):
    raise NotImplementedError("write your pallas kernel here")


# speedup vs baseline: 1.4540x; 1.4540x over previous
import functools

import jax
import jax.numpy as jnp
from jax import lax
from jax.experimental import pallas as pl
from jax.experimental.pallas import tpu as pltpu
from jax.experimental.pallas import tpu_sc as plsc

D = 64
C = 128


def _sc_embed(words_flat, feats_flat, word_table, feat_table):
    n = words_flat.shape[0]
    info = plsc.get_sparse_core_info()
    nc, ns = info.num_cores, info.num_subcores
    nw = nc * ns
    per_w = n // nw
    chunks = per_w // C
    assert per_w * nw == n and chunks * C == per_w

    mesh = plsc.VectorSubcoreMesh(core_axis_name="c", subcore_axis_name="s")

    @functools.partial(
        pl.kernel,
        out_type=(
            jax.ShapeDtypeStruct((n, D), jnp.float32),
            jax.ShapeDtypeStruct((n, D), jnp.float32),
        ),
        mesh=mesh,
        compiler_params=pltpu.CompilerParams(use_tc_tiling_on_sc=False),
        scratch_types=[
            pltpu.VMEM((C,), jnp.int32),
            pltpu.VMEM((C,), jnp.int32),
            pltpu.VMEM((C, D), jnp.float32),
            pltpu.VMEM((C, D), jnp.float32),
            pltpu.SemaphoreType.DMA,
            pltpu.SemaphoreType.DMA,
        ],
    )
    def k(words_hbm, feats_hbm, wtab_hbm, ftab_hbm, sum_out, feat_out,
          idxw, idxf, rw, rf, sem_w, sem_f):
        wid = lax.axis_index("s") * nc + lax.axis_index("c")
        base = wid * per_w

        def chunk(i, carry):
            off = base + i * C
            pltpu.sync_copy(words_hbm.at[pl.ds(off, C)], idxw)
            pltpu.sync_copy(feats_hbm.at[pl.ds(off, C)], idxf)
            cw = pltpu.async_copy(wtab_hbm.at[idxw], rw, sem_w)
            cf = pltpu.async_copy(ftab_hbm.at[idxf], rf, sem_f)
            cf.wait()
            pltpu.sync_copy(rf, feat_out.at[pl.ds(off, C)])
            cw.wait()

            def row(t, rcarry):
                for kk in range(D // 16):
                    s = pl.ds(kk * 16, 16)
                    rw[t, s] = rw[t, s] + rf[t, s]
                return rcarry

            lax.fori_loop(0, C, row, 0)
            pltpu.sync_copy(rw, sum_out.at[pl.ds(off, C)])
            return carry

        lax.fori_loop(0, chunks, chunk, 0)

    return k(words_flat, feats_flat, word_table, feat_table)


def _mask_body(w_ref, m_ref):
    m_ref[...] = w_ref[...] != 0


def kernel(words, feats, word_table, feat_table):
    batch, seq = words.shape
    n = batch * seq
    s, f = _sc_embed(words.reshape(n), feats.reshape(n), word_table, feat_table)
    mask = pl.pallas_call(
        _mask_body,
        out_shape=jax.ShapeDtypeStruct((batch, seq), jnp.bool_),
    )(words)
    return (s.reshape(batch, seq, D), f.reshape(batch, seq, D), mask, seq)


# speedup vs baseline: 1.5168x; 1.0432x over previous
import functools

import jax
import jax.numpy as jnp
from jax import lax
from jax.experimental import pallas as pl
from jax.experimental.pallas import tpu as pltpu
from jax.experimental.pallas import tpu_sc as plsc

D = 64
C = 128
NB = 4


def _sc_embed(words3, feats3, word_table, feat_table):
    nw, chunks, c = words3.shape
    assert c == C and chunks % NB == 0 and chunks >= 2 * NB
    n = nw * chunks * C
    per_w = chunks * C
    rounds = chunks // NB

    mesh = plsc.VectorSubcoreMesh(core_axis_name="c", subcore_axis_name="s")
    info = plsc.get_sparse_core_info()
    nc = info.num_cores
    assert nw == nc * info.num_subcores

    @functools.partial(
        pl.kernel,
        out_type=(
            jax.ShapeDtypeStruct((n, D), jnp.float32),
            jax.ShapeDtypeStruct((n, D), jnp.float32),
        ),
        mesh=mesh,
        compiler_params=pltpu.CompilerParams(use_tc_tiling_on_sc=False),
        scratch_types=[
            pltpu.VMEM((chunks, C), jnp.int32),
            pltpu.VMEM((chunks, C), jnp.int32),
            pltpu.VMEM((NB, C, D), jnp.float32),
            pltpu.VMEM((NB, C, D), jnp.float32),
        ]
        + [pltpu.SemaphoreType.DMA] * (2 * NB + 1),
    )
    def k(words_hbm, feats_hbm, wtab_hbm, ftab_hbm, sum_out, feat_out,
          idxw, idxf, rw, rf, *sems):
        gsem = sems[:NB]
        wsem = sems[NB:2 * NB]
        isem = sems[2 * NB]
        wid = lax.axis_index("s") * nc + lax.axis_index("c")
        base = wid * per_w

        ci = pltpu.async_copy(words_hbm.at[wid], idxw, isem)
        cf = pltpu.async_copy(feats_hbm.at[wid], idxf, isem)
        ci.wait()
        cf.wait()

        def start_gather(cl, b):
            pltpu.async_copy(wtab_hbm.at[idxw.at[cl]], rw.at[b], gsem[b])
            pltpu.async_copy(ftab_hbm.at[idxf.at[cl]], rf.at[b], gsem[b])

        def drain_gather(b):
            pltpu.make_async_copy(wtab_hbm.at[pl.ds(0, C)], rw.at[b],
                                  gsem[b]).wait()
            pltpu.make_async_copy(wtab_hbm.at[pl.ds(0, C)], rf.at[b],
                                  gsem[b]).wait()

        def drain_write(b):
            pltpu.make_async_copy(rw.at[b], sum_out.at[pl.ds(0, C)],
                                  wsem[b]).wait()
            pltpu.make_async_copy(rf.at[b], feat_out.at[pl.ds(0, C)],
                                  wsem[b]).wait()

        start_gather(0, 0)
        start_gather(1, 1)

        def round_body(r, carry):
            for b in range(NB):
                i = r * NB + b
                bg = (b + 2) % NB

                @pl.when(i + 2 < chunks)
                def _():
                    @pl.when(i >= 2)
                    def _():
                        drain_write(bg)
                    start_gather(i + 2, bg)

                drain_gather(b)

                def row(t, rcarry):
                    for kk in range(D // 16):
                        s = pl.ds(kk * 16, 16)
                        rw[b, t, s] = rw[b, t, s] + rf[b, t, s]
                    return rcarry

                lax.fori_loop(0, C, row, 0, unroll=2)
                off = base + i * C
                pltpu.async_copy(rw.at[b], sum_out.at[pl.ds(off, C)], wsem[b])
                pltpu.async_copy(rf.at[b], feat_out.at[pl.ds(off, C)], wsem[b])
            return carry

        lax.fori_loop(0, rounds, round_body, 0)
        for b in range(NB):
            drain_write(b)

    return k(words3, feats3, word_table, feat_table)


def _mask_body(w_ref, m_ref):
    m_ref[...] = w_ref[...] != 0


def kernel(words, feats, word_table, feat_table):
    batch, seq = words.shape
    n = batch * seq
    nw = 32
    per_w = n // nw
    chunks = per_w // C
    s, f = _sc_embed(
        words.reshape(nw, chunks, C),
        feats.reshape(nw, chunks, C),
        word_table,
        feat_table,
    )
    mask = pl.pallas_call(
        _mask_body,
        out_shape=jax.ShapeDtypeStruct((batch, seq), jnp.bool_),
    )(words)
    return (s.reshape(batch, seq, D), f.reshape(batch, seq, D), mask, seq)


# speedup vs baseline: 1.7014x; 1.1217x over previous
import functools

import jax
import jax.numpy as jnp
from jax import lax
from jax.experimental import pallas as pl
from jax.experimental.pallas import tpu as pltpu
from jax.experimental.pallas import tpu_sc as plsc

D = 64
C = 128
NB = 8
KA = 5
TB = 8


def _sc_gather(words3, word_table):
    nw, chunks, c = words3.shape
    assert c == C and chunks % NB == 0 and chunks >= 2 * NB
    n = nw * chunks * C
    per_w = chunks * C
    rounds = chunks // NB

    mesh = plsc.VectorSubcoreMesh(core_axis_name="c", subcore_axis_name="s")
    info = plsc.get_sparse_core_info()
    nc = info.num_cores
    assert nw == nc * info.num_subcores

    @functools.partial(
        pl.kernel,
        out_type=jax.ShapeDtypeStruct((n, D), jnp.float32),
        mesh=mesh,
        compiler_params=pltpu.CompilerParams(use_tc_tiling_on_sc=False),
        scratch_types=[
            pltpu.VMEM((chunks, C), jnp.int32),
            pltpu.VMEM((NB, C, D), jnp.float32),
        ]
        + [pltpu.SemaphoreType.DMA] * (2 * NB + 1),
    )
    def k(words_hbm, wtab_hbm, w_out, idxw, rw, *sems):
        gsem = sems[:NB]
        wsem = sems[NB:2 * NB]
        isem = sems[2 * NB]
        wid = lax.axis_index("s") * nc + lax.axis_index("c")
        base = wid * per_w

        pltpu.async_copy(words_hbm.at[wid], idxw, isem).wait()

        def start_gather(cl, b):
            pltpu.async_copy(wtab_hbm.at[idxw.at[cl]], rw.at[b], gsem[b])

        def drain_gather(b):
            pltpu.make_async_copy(wtab_hbm.at[pl.ds(0, C)], rw.at[b],
                                  gsem[b]).wait()

        def drain_write(b):
            pltpu.make_async_copy(rw.at[b], w_out.at[pl.ds(0, C)],
                                  wsem[b]).wait()

        for j in range(KA):
            start_gather(j, j)

        def round_body(r, carry):
            for b in range(NB):
                i = r * NB + b
                bg = (b + KA) % NB

                @pl.when(i + KA < chunks)
                def _():
                    @pl.when(i >= NB - KA)
                    def _():
                        drain_write(bg)
                    start_gather(i + KA, bg)

                drain_gather(b)
                off = base + i * C
                pltpu.async_copy(rw.at[b], w_out.at[pl.ds(off, C)], wsem[b])
            return carry

        lax.fori_loop(0, rounds, round_body, 0)
        for b in range(NB):
            drain_write(b)

    return k(words3, word_table)


def _epilogue_body(w_ref, words_ref, feats_ref, ftab_ref,
                   sum_ref, feat_ref, mask_ref):
    tb, seq = words_ref.shape
    t = tb * seq
    w2 = w_ref[...]
    left = w2[:, :D].reshape(t // 2, 1, D)
    right = w2[:, D:].reshape(t // 2, 1, D)
    w64 = jnp.concatenate([left, right], axis=1).reshape(t, D)
    f1 = feats_ref[0]
    onehot_t = (lax.broadcasted_iota(jnp.int32, (2 * D, t), 0) == f1).astype(
        jnp.float32)
    fe = jnp.dot(ftab_ref[...], onehot_t,
                 preferred_element_type=jnp.float32).T
    feat_ref[...] = fe.reshape(tb, seq, D)
    sum_ref[...] = (w64 + fe).reshape(tb, seq, D)
    mask_ref[...] = words_ref[...] != 0


def kernel(words, feats, word_table, feat_table):
    batch, seq = words.shape
    n = batch * seq
    nw = 32
    chunks = n // (nw * C)
    w_rows = _sc_gather(words.reshape(nw, chunks, C), word_table)

    ftab_t = jnp.pad(feat_table, ((0, 2 * D - feat_table.shape[0]), (0, 0))).T
    s, f, mask = pl.pallas_call(
        _epilogue_body,
        grid=(batch // TB,),
        in_specs=[
            pl.BlockSpec((TB * seq // 2, 2 * D), lambda i: (i, 0)),
            pl.BlockSpec((TB, seq), lambda i: (i, 0)),
            pl.BlockSpec((1, 1, TB * seq), lambda i: (i, 0, 0)),
            pl.BlockSpec((D, 2 * D), lambda i: (0, 0)),
        ],
        out_specs=[
            pl.BlockSpec((TB, seq, D), lambda i: (i, 0, 0)),
            pl.BlockSpec((TB, seq, D), lambda i: (i, 0, 0)),
            pl.BlockSpec((TB, seq), lambda i: (i, 0)),
        ],
        out_shape=[
            jax.ShapeDtypeStruct((batch, seq, D), jnp.float32),
            jax.ShapeDtypeStruct((batch, seq, D), jnp.float32),
            jax.ShapeDtypeStruct((batch, seq), jnp.bool_),
        ],
    )(w_rows.reshape(n // 2, 2 * D), words,
      feats.reshape(batch // TB, 1, TB * seq), ftab_t)
    return (s, f, mask, seq)
